# baseline (speedup 1.0000x reference)
import functools

import jax
import jax.numpy as jnp
from jax import lax
from jax.experimental import pallas as pl
from jax.experimental.pallas import tpu as pltpu
from jax.experimental.pallas import tpu_sc as plsc

N = 4096
EDIM = 128
WFEAT = 128
SLOPE = 0.01
BLK = 256
NBLK = N // BLK
NEG = -1e9


def _sc_gather(emb, idx):
    info = plsc.get_sparse_core_info()
    nc, ns = info.num_cores, info.num_subcores
    nw = nc * ns
    b_per_w = N // nw
    mesh = plsc.VectorSubcoreMesh(core_axis_name="c", subcore_axis_name="s")

    @functools.partial(
        pl.kernel,
        mesh=mesh,
        out_type=jax.ShapeDtypeStruct((N, EDIM), jnp.float32),
        scratch_types=[
            pltpu.VMEM((b_per_w,), jnp.int32),
            pltpu.VMEM((b_per_w, EDIM), jnp.float32),
            pltpu.SemaphoreType.DMA,
        ],
    )
    def gather_k(table_hbm, idx_hbm, out_hbm, idx_v, rows_v, sem):
        wid = lax.axis_index("s") * nc + lax.axis_index("c")
        base = wid * b_per_w
        pltpu.sync_copy(idx_hbm.at[pl.ds(base, b_per_w)], idx_v)
        pltpu.async_copy(table_hbm.at[idx_v], rows_v, sem).wait()
        pltpu.sync_copy(rows_v, out_hbm.at[pl.ds(base, b_per_w)])

    return gather_k(emb, idx)


def _scores(i, x_ref, W_ref, a1_ref, a2_ref, adj_ref, h_ref, f2_ref, m2_ref):
    @pl.when(i == 0)
    def _init():
        h = jnp.dot(x_ref[...], W_ref[...], preferred_element_type=jnp.float32)
        h_ref[...] = h
        f2row = lax.dot_general(
            a2_ref[...], h, (((1,), (1,)), ((), ())),
            preferred_element_type=jnp.float32)
        f2_ref[...] = f2row
        m2_ref[...] = jnp.max(f2row, axis=1, keepdims=True)

    h_blk = h_ref[pl.ds(i * BLK, BLK), :]
    f1 = jnp.dot(h_blk, a1_ref[...], preferred_element_type=jnp.float32)
    m = f1 + m2_ref[...]
    m = jnp.maximum(m, SLOPE * m)
    e = f1 + f2_ref[...]
    e = jnp.maximum(e, SLOPE * e)
    p = jnp.where(adj_ref[...], jnp.exp(e - m), 0.0)
    ones = f2_ref[...] * 0.0 + 1.0
    s = lax.dot_general(p, ones, (((1,), (1,)), ((), ())),
                        preferred_element_type=jnp.float32)
    return p * (1.0 / s)


def _gat1_body(x_ref, W_ref, a1_ref, a2_ref, adj_ref, att_ref, out_ref,
               h_ref, f2_ref, m2_ref):
    i = pl.program_id(0)
    att = _scores(i, x_ref, W_ref, a1_ref, a2_ref, adj_ref, h_ref, f2_ref,
                  m2_ref)
    att_ref[...] = att
    o = jnp.dot(att, h_ref[...], preferred_element_type=jnp.float32)
    out_ref[...] = jnp.where(o > 0, o, jnp.exp(o) - 1.0)


def _gat2_body(x_ref, W_ref, a1_ref, a2_ref, adj_ref, att_ref, dsum_ref,
               h_ref, f2_ref, m2_ref):
    i = pl.program_id(0)
    att = _scores(i, x_ref, W_ref, a1_ref, a2_ref, adj_ref, h_ref, f2_ref,
                  m2_ref)
    att_ref[...] = att
    o = jnp.dot(att, h_ref[...], preferred_element_type=jnp.float32)
    doc = jnp.where(o > 0, o, jnp.exp(o) - 1.0) + x_ref[pl.ds(i * BLK, BLK), :]

    @pl.when(i == 0)
    def _zero():
        dsum_ref[...] = jnp.zeros_like(dsum_ref)

    dsum_ref[...] = dsum_ref[...] + jnp.sum(doc, axis=0, keepdims=True)

    @pl.when(i == NBLK - 1)
    def _scale():
        dsum_ref[...] = dsum_ref[...] * (1.0 / N)


def _gat_specs():
    in_specs = [
        pl.BlockSpec((N, EDIM), lambda i: (0, 0)),
        pl.BlockSpec((EDIM, WFEAT), lambda i: (0, 0)),
        pl.BlockSpec((WFEAT, 1), lambda i: (0, 0)),
        pl.BlockSpec((1, WFEAT), lambda i: (0, 0)),
        pl.BlockSpec((BLK, N), lambda i: (i, 0)),
    ]
    scratch = [
        pltpu.VMEM((N, WFEAT), jnp.float32),
        pltpu.VMEM((1, N), jnp.float32),
        pltpu.VMEM((1, 1), jnp.float32),
    ]
    return in_specs, scratch


def _gat_layer1(x, adj, W, a):
    a1 = a[:WFEAT].reshape(WFEAT, 1)
    a2 = a[WFEAT:].reshape(1, WFEAT)
    in_specs, scratch = _gat_specs()
    att, out = pl.pallas_call(
        _gat1_body,
        grid=(NBLK,),
        in_specs=in_specs,
        out_specs=[
            pl.BlockSpec((BLK, N), lambda i: (i, 0)),
            pl.BlockSpec((BLK, WFEAT), lambda i: (i, 0)),
        ],
        out_shape=[
            jax.ShapeDtypeStruct((N, N), jnp.float32),
            jax.ShapeDtypeStruct((N, WFEAT), jnp.float32),
        ],
        scratch_shapes=scratch,
        compiler_params=pltpu.CompilerParams(
            dimension_semantics=("arbitrary",)),
    )(x, W, a1, a2, adj)
    return att, out


def _gat_layer2(x, adj, W, a):
    a1 = a[:WFEAT].reshape(WFEAT, 1)
    a2 = a[WFEAT:].reshape(1, WFEAT)
    in_specs, scratch = _gat_specs()
    att, dsum = pl.pallas_call(
        _gat2_body,
        grid=(NBLK,),
        in_specs=in_specs,
        out_specs=[
            pl.BlockSpec((BLK, N), lambda i: (i, 0)),
            pl.BlockSpec((1, WFEAT), lambda i: (0, 0)),
        ],
        out_shape=[
            jax.ShapeDtypeStruct((N, N), jnp.float32),
            jax.ShapeDtypeStruct((1, WFEAT), jnp.float32),
        ],
        scratch_shapes=scratch,
        compiler_params=pltpu.CompilerParams(
            dimension_semantics=("arbitrary",)),
    )(x, W, a1, a2, adj)
    return att, dsum


def kernel(inDoc, adj0, adj1, emb, W_s, a_s, W_d, a_d):
    words = _sc_gather(emb, inDoc.astype(jnp.int32))
    sattention, words1 = _gat_layer1(words, adj0, W_s, a_s)
    dattention, dsum = _gat_layer2(words1, adj1, W_d, a_d)
    return (dsum[0], sattention, dattention)


# speedup vs baseline: 1.0429x; 1.0429x over previous
import functools

import jax
import jax.numpy as jnp
from jax import lax
from jax.experimental import pallas as pl
from jax.experimental.pallas import tpu as pltpu
from jax.experimental.pallas import tpu_sc as plsc

N = 4096
EDIM = 128
WFEAT = 128
SLOPE = 0.01
BLK = 512
NBLK = N // BLK
NEG = -1e9


def _sc_gather(emb, idx):
    info = plsc.get_sparse_core_info()
    nc, ns = info.num_cores, info.num_subcores
    nw = nc * ns
    b_per_w = N // nw
    mesh = plsc.VectorSubcoreMesh(core_axis_name="c", subcore_axis_name="s")

    @functools.partial(
        pl.kernel,
        mesh=mesh,
        out_type=jax.ShapeDtypeStruct((N, EDIM), jnp.float32),
        scratch_types=[
            pltpu.VMEM((b_per_w,), jnp.int32),
            pltpu.VMEM((b_per_w, EDIM), jnp.float32),
            pltpu.SemaphoreType.DMA,
        ],
    )
    def gather_k(table_hbm, idx_hbm, out_hbm, idx_v, rows_v, sem):
        wid = lax.axis_index("s") * nc + lax.axis_index("c")
        base = wid * b_per_w
        pltpu.sync_copy(idx_hbm.at[pl.ds(base, b_per_w)], idx_v)
        pltpu.async_copy(table_hbm.at[idx_v], rows_v, sem).wait()
        pltpu.sync_copy(rows_v, out_hbm.at[pl.ds(base, b_per_w)])

    return gather_k(emb, idx)


def _scores(i, x_ref, W_ref, a1_ref, a2_ref, adj_ref, h_ref, f2_ref, m2_ref):
    @pl.when(i == 0)
    def _init():
        h = jnp.dot(x_ref[...], W_ref[...], preferred_element_type=jnp.float32)
        h_ref[...] = h
        f2row = lax.dot_general(
            a2_ref[...], h, (((1,), (1,)), ((), ())),
            preferred_element_type=jnp.float32)
        f2_ref[...] = f2row
        m2_ref[...] = jnp.max(f2row, axis=1, keepdims=True)

    h_blk = h_ref[pl.ds(i * BLK, BLK), :]
    f1 = jnp.dot(h_blk, a1_ref[...], preferred_element_type=jnp.float32)
    m = f1 + m2_ref[...]
    m = jnp.maximum(m, SLOPE * m)
    e = f1 + f2_ref[...]
    e = jnp.maximum(e, SLOPE * e)
    p = jnp.where(adj_ref[...], jnp.exp(e - m), 0.0)
    ones = f2_ref[...] * 0.0 + 1.0
    s = lax.dot_general(p, ones, (((1,), (1,)), ((), ())),
                        preferred_element_type=jnp.float32)
    return p * (1.0 / s)


def _gat1_body(x_ref, W_ref, a1_ref, a2_ref, adj_ref, att_ref, out_ref,
               h_ref, f2_ref, m2_ref):
    i = pl.program_id(0)
    att = _scores(i, x_ref, W_ref, a1_ref, a2_ref, adj_ref, h_ref, f2_ref,
                  m2_ref)
    att_ref[...] = att
    o = jnp.dot(att, h_ref[...], preferred_element_type=jnp.float32)
    out_ref[...] = jnp.where(o > 0, o, jnp.exp(o) - 1.0)


def _gat2_body(x_ref, W_ref, a1_ref, a2_ref, adj_ref, att_ref, dsum_ref,
               h_ref, f2_ref, m2_ref):
    i = pl.program_id(0)
    att = _scores(i, x_ref, W_ref, a1_ref, a2_ref, adj_ref, h_ref, f2_ref,
                  m2_ref)
    att_ref[...] = att
    o = jnp.dot(att, h_ref[...], preferred_element_type=jnp.float32)
    doc = jnp.where(o > 0, o, jnp.exp(o) - 1.0) + x_ref[pl.ds(i * BLK, BLK), :]

    @pl.when(i == 0)
    def _zero():
        dsum_ref[...] = jnp.zeros_like(dsum_ref)

    dsum_ref[...] = dsum_ref[...] + jnp.sum(doc, axis=0, keepdims=True)

    @pl.when(i == NBLK - 1)
    def _scale():
        dsum_ref[...] = dsum_ref[...] * (1.0 / N)


def _gat_specs():
    in_specs = [
        pl.BlockSpec((N, EDIM), lambda i: (0, 0)),
        pl.BlockSpec((EDIM, WFEAT), lambda i: (0, 0)),
        pl.BlockSpec((WFEAT, 1), lambda i: (0, 0)),
        pl.BlockSpec((1, WFEAT), lambda i: (0, 0)),
        pl.BlockSpec((BLK, N), lambda i: (i, 0)),
    ]
    scratch = [
        pltpu.VMEM((N, WFEAT), jnp.float32),
        pltpu.VMEM((1, N), jnp.float32),
        pltpu.VMEM((1, 1), jnp.float32),
    ]
    return in_specs, scratch


def _gat_layer1(x, adj, W, a):
    a1 = a[:WFEAT].reshape(WFEAT, 1)
    a2 = a[WFEAT:].reshape(1, WFEAT)
    in_specs, scratch = _gat_specs()
    att, out = pl.pallas_call(
        _gat1_body,
        grid=(NBLK,),
        in_specs=in_specs,
        out_specs=[
            pl.BlockSpec((BLK, N), lambda i: (i, 0)),
            pl.BlockSpec((BLK, WFEAT), lambda i: (i, 0)),
        ],
        out_shape=[
            jax.ShapeDtypeStruct((N, N), jnp.float32),
            jax.ShapeDtypeStruct((N, WFEAT), jnp.float32),
        ],
        scratch_shapes=scratch,
        compiler_params=pltpu.CompilerParams(
            dimension_semantics=("arbitrary",)),
    )(x, W, a1, a2, adj)
    return att, out


def _gat_layer2(x, adj, W, a):
    a1 = a[:WFEAT].reshape(WFEAT, 1)
    a2 = a[WFEAT:].reshape(1, WFEAT)
    in_specs, scratch = _gat_specs()
    att, dsum = pl.pallas_call(
        _gat2_body,
        grid=(NBLK,),
        in_specs=in_specs,
        out_specs=[
            pl.BlockSpec((BLK, N), lambda i: (i, 0)),
            pl.BlockSpec((1, WFEAT), lambda i: (0, 0)),
        ],
        out_shape=[
            jax.ShapeDtypeStruct((N, N), jnp.float32),
            jax.ShapeDtypeStruct((1, WFEAT), jnp.float32),
        ],
        scratch_shapes=scratch,
        compiler_params=pltpu.CompilerParams(
            dimension_semantics=("arbitrary",)),
    )(x, W, a1, a2, adj)
    return att, dsum


def kernel(inDoc, adj0, adj1, emb, W_s, a_s, W_d, a_d):
    words = _sc_gather(emb, inDoc.astype(jnp.int32))
    sattention, words1 = _gat_layer1(words, adj0, W_s, a_s)
    dattention, dsum = _gat_layer2(words1, adj1, W_d, a_d)
    return (dsum[0], sattention, dattention)
